# baseline (speedup 1.0000x reference)
import jax
import jax.numpy as jnp
from jax import lax
from jax.experimental import pallas as pl
from jax.experimental.pallas import tpu as pltpu
from jax.experimental.pallas import tpu_sc as plsc

N = 50000
E = 800000
H = 64
NB = 10
PER = N // NB

NHALF = N // 2
AGG_ROWS = NHALF + 200
K = 80
GROUP = 2000
CPG = GROUP // K
EPT = E // 16
NGRP = EPT // GROUP
NZCH = AGG_ROWS // K
WBCH = 200
NWB = NHALF // WBCH


def _h0_matmul(xa, W, b):
    R = 5000

    def kern(xa_ref, W_ref, b_ref, o_ref):
        o_ref[...] = jnp.maximum(
            jnp.dot(xa_ref[...], W_ref[...], preferred_element_type=jnp.float32)
            + b_ref[...][None, :], 0.0)

    return pl.pallas_call(
        kern,
        grid=(N // R,),
        in_specs=[pl.BlockSpec((R, 4), lambda i: (i, 0)),
                  pl.BlockSpec((4, H), lambda i: (0, 0)),
                  pl.BlockSpec((H,), lambda i: (0,))],
        out_specs=pl.BlockSpec((R, H), lambda i: (i, 0)),
        out_shape=jax.ShapeDtypeStruct((N, H), jnp.float32),
    )(xa, W, b)


def _edge_mlp(ea_e, ea_o, We1, be1, We2, be2, Wl, bl):
    R = 5000
    L = Wl.shape[0]

    def kern(eae_ref, eao_ref, We1_ref, be1_ref, We2_ref, be2_ref,
             Wl_ref, bl_ref, *outs):
        def emb(ref):
            t = jnp.maximum(
                jnp.dot(ref[...], We1_ref[...], preferred_element_type=jnp.float32)
                + be1_ref[...][None, :], 0.0)
            return jnp.dot(t, We2_ref[...], preferred_element_type=jnp.float32) \
                + be2_ref[...][None, :]
        e_e = emb(eae_ref)
        e_o = emb(eao_ref)
        for l in range(L):
            le = jnp.dot(e_e, Wl_ref[l], preferred_element_type=jnp.float32) \
                + bl_ref[l][None, :]
            lo = jnp.dot(e_o, Wl_ref[l], preferred_element_type=jnp.float32) \
                + bl_ref[l][None, :]
            outs[l][...] = jnp.concatenate([le, lo], axis=1)

    return pl.pallas_call(
        kern,
        grid=(E // 2 // R,),
        in_specs=[pl.BlockSpec((R, 4), lambda i: (i, 0)),
                  pl.BlockSpec((R, 4), lambda i: (i, 0)),
                  pl.BlockSpec((4, H), lambda i: (0, 0)),
                  pl.BlockSpec((H,), lambda i: (0,)),
                  pl.BlockSpec((H, H), lambda i: (0, 0)),
                  pl.BlockSpec((H,), lambda i: (0,)),
                  pl.BlockSpec((L, H, H), lambda i: (0, 0, 0)),
                  pl.BlockSpec((L, H), lambda i: (0, 0))],
        out_specs=[pl.BlockSpec((R, 2 * H), lambda i: (i, 0))] * L,
        out_shape=[jax.ShapeDtypeStruct((E // 2, 2 * H), jnp.float32)] * L,
    )(ea_e, ea_o, We1, be1, We2, be2, Wl, bl)


def _node_update(h, aggr, Wa, ba, Wb, bb, g, bt):
    R = 5000

    def kern(h_ref, a_ref, Wa_ref, ba_ref, Wb_ref, bb_ref, g_ref, bt_ref, o_ref):
        hv = h_ref[...]
        z = hv + a_ref[...]
        z1 = jnp.maximum(
            jnp.dot(z, Wa_ref[...], preferred_element_type=jnp.float32)
            + ba_ref[...][None, :], 0.0)
        z2 = jnp.dot(z1, Wb_ref[...], preferred_element_type=jnp.float32) \
            + bb_ref[...][None, :]
        mu = jnp.mean(z2, axis=1, keepdims=True)
        d = z2 - mu
        var = jnp.mean(d * d, axis=1, keepdims=True)
        zn = d * lax.rsqrt(var + 1e-5) * g_ref[...][None, :] + bt_ref[...][None, :]
        o_ref[...] = hv + jnp.maximum(zn, 0.0)

    return pl.pallas_call(
        kern,
        grid=(N // R,),
        in_specs=[pl.BlockSpec((R, H), lambda i: (i, 0)),
                  pl.BlockSpec((R, H), lambda i: (i, 0)),
                  pl.BlockSpec((H, H), lambda i: (0, 0)),
                  pl.BlockSpec((H,), lambda i: (0,)),
                  pl.BlockSpec((H, H), lambda i: (0, 0)),
                  pl.BlockSpec((H,), lambda i: (0,)),
                  pl.BlockSpec((H,), lambda i: (0,)),
                  pl.BlockSpec((H,), lambda i: (0,))],
        out_specs=pl.BlockSpec((R, H), lambda i: (i, 0)),
        out_shape=jax.ShapeDtypeStruct((N, H), jnp.float32),
    )(h, aggr, Wa, ba, Wb, bb, g, bt)


def _pool_head(h, Wh1, bh1, Wh2, bh2):
    def kern(h_ref, W1_ref, b1_ref, W2_ref, b2_ref, o_ref):
        gs = []
        for b in range(NB):
            seg = h_ref[pl.ds(b * PER, PER), :]
            gs.append(jnp.sum(seg, axis=0) / float(PER))
        g = jnp.stack(gs, axis=0)
        q1 = jnp.maximum(
            jnp.dot(g, W1_ref[...], preferred_element_type=jnp.float32)
            + b1_ref[...][None, :], 0.0)
        q = jnp.dot(q1, W2_ref[...], preferred_element_type=jnp.float32) \
            + b2_ref[...][None, :]
        o_ref[...] = q[:, 0]

    return pl.pallas_call(
        kern,
        out_shape=jax.ShapeDtypeStruct((NB,), jnp.float32),
    )(h, Wh1, bh1, Wh2, bh2)


def _sc_aggr_body(src_ref, dst_ref, h_ref, el_ref, out_ref,
                  srcg, dstg, idxb, hb0, hb1, eb0, eb1, aggr_sh,
                  gsem0, gsem1):
    c_ax = lax.axis_index("c")
    s_ax = lax.axis_index("s")
    base_node = c_ax * NHALF

    hbs = (hb0, hb1)
    ebs = (eb0, eb1)
    sems = (gsem0, gsem1)

    def zrow(r, carry):
        for q in range(H // 16):
            hb0[r, pl.ds(q * 16, 16)] = jnp.zeros((16,), jnp.float32)
        return carry
    lax.fori_loop(0, K, zrow, 0)
    nz = jnp.where(s_ax < (NZCH % 16), NZCH // 16 + 1, NZCH // 16)

    def zchunk(i, carry):
        cid = s_ax + 16 * i
        pltpu.sync_copy(hb0.at[:, :], aggr_sh.at[pl.ds(cid * K, K), :])
        return carry
    lax.fori_loop(0, nz, zchunk, 0)
    plsc.subcore_barrier()

    tbase = s_ax * EPT

    def fire(go, ci, p):
        pltpu.async_copy(h_ref.at[srcg.at[pl.ds(ci * K, K)]], hbs[p], sems[p])
        pltpu.async_copy(
            el_ref.at[pl.ds((go + ci * K) // 2, K // 2), :], ebs[p], sems[p])

    def proc(go, ci, p):
        pltpu.make_async_copy(
            h_ref.at[srcg.at[pl.ds(ci * K, K)]], hbs[p], sems[p]).wait()
        pltpu.make_async_copy(
            el_ref.at[pl.ds((go + ci * K) // 2, K // 2), :], ebs[p], sems[p]).wait()

        def ib(v, icarry):
            d = dstg[pl.ds(ci * K + v * 16, 16)]
            dl = d - base_node
            ok = (dl >= 0) & (dl < NHALF)
            idxb[pl.ds(v * 16, 16)] = jnp.where(ok, dl, NHALF)
            return icarry
        lax.fori_loop(0, K // 16, ib, 0)
        hb, eb = hbs[p], ebs[p]

        def mrow(rp, mcarry):
            for half in range(2):
                for q in range(H // 16):
                    hb[2 * rp + half, pl.ds(q * 16, 16)] = jnp.maximum(
                        hb[2 * rp + half, pl.ds(q * 16, 16)]
                        + eb[rp, pl.ds(half * H + q * 16, 16)], 0.0)
            return mcarry
        lax.fori_loop(0, K // 2, mrow, 0)
        pltpu.sync_copy(hb, aggr_sh.at[idxb], add=True)

    def group(g, carry):
        go = tbase + g * GROUP
        pltpu.sync_copy(src_ref.at[pl.ds(go, GROUP)], srcg)
        pltpu.sync_copy(dst_ref.at[pl.ds(go, GROUP)], dstg)
        fire(go, 0, 0)

        def pair(kk, pcarry):
            cA = 2 * kk
            fire(go, cA + 1, 1)
            proc(go, cA, 0)
            fire(go, cA + 2, 0)
            proc(go, cA + 1, 1)
            return pcarry
        lax.fori_loop(0, (CPG - 3) // 2, pair, 0)
        fire(go, CPG - 2, 1)
        proc(go, CPG - 3, 0)
        fire(go, CPG - 1, 0)
        proc(go, CPG - 2, 1)
        proc(go, CPG - 1, 0)
        return carry
    lax.fori_loop(0, NGRP, group, 0)

    plsc.subcore_barrier()
    nw = jnp.where(s_ax < (NWB % 16), NWB // 16 + 1, NWB // 16)

    def wchunk(i, carry):
        cid = s_ax + 16 * i
        pltpu.sync_copy(aggr_sh.at[pl.ds(cid * WBCH, WBCH), :],
                        out_ref.at[pl.ds(base_node + cid * WBCH, WBCH), :])
        return carry
    lax.fori_loop(0, nw, wchunk, 0)


def _sc_aggregate(src, dst, h, elin):
    mesh = plsc.VectorSubcoreMesh(core_axis_name="c", subcore_axis_name="s",
                                  num_cores=2, num_subcores=16)
    f = pl.kernel(
        _sc_aggr_body,
        out_type=jax.ShapeDtypeStruct((N, H), jnp.float32),
        mesh=mesh,
        scratch_types=[
            pltpu.VMEM((GROUP,), jnp.int32),
            pltpu.VMEM((GROUP,), jnp.int32),
            pltpu.VMEM((K,), jnp.int32),
            pltpu.VMEM((K, H), jnp.float32),
            pltpu.VMEM((K, H), jnp.float32),
            pltpu.VMEM((K // 2, 2 * H), jnp.float32),
            pltpu.VMEM((K // 2, 2 * H), jnp.float32),
            pltpu.VMEM_SHARED((AGG_ROWS, H), jnp.float32),
            pltpu.SemaphoreType.DMA,
            pltpu.SemaphoreType.DMA,
        ],
        compiler_params=pltpu.CompilerParams(use_tc_tiling_on_sc=False),
    )
    return f(src, dst, h, elin)


def kernel(x, edge_index, edge_attr, batch_ids, ptr, a,
           W_np, b_np, We1, be1, We2, be2,
           conv_Wl, conv_bl, conv_Wa, conv_ba, conv_Wb, conv_bb,
           ln_g, ln_b, Wh1, bh1, Wh2, bh2):
    L = conv_Wl.shape[0]
    xa = jnp.concatenate([x, a.reshape(-1, 1)], axis=1)
    h = _h0_matmul(xa, W_np, b_np)
    elins = _edge_mlp(edge_attr[0::2], edge_attr[1::2],
                      We1, be1, We2, be2, conv_Wl, conv_bl)
    src, dst = edge_index[0], edge_index[1]
    for l in range(L):
        aggr = _sc_aggregate(src, dst, h, elins[l])
        h = _node_update(h, aggr, conv_Wa[l], conv_ba[l],
                         conv_Wb[l], conv_bb[l], ln_g[l], ln_b[l])
    return _pool_head(h, Wh1, bh1, Wh2, bh2)


# speedup vs baseline: 1.0356x; 1.0356x over previous
import jax
import jax.numpy as jnp
from jax import lax
from jax.experimental import pallas as pl
from jax.experimental.pallas import tpu as pltpu
from jax.experimental.pallas import tpu_sc as plsc

N = 50000
E = 800000
H = 64
NB = 10
PER = N // NB

NHALF = N // 2
AGG_ROWS = NHALF + 200
K = 80
GROUP = 2000
CPG = GROUP // K
EPT = E // 16
NGRP = EPT // GROUP
NZCH = AGG_ROWS // K
WBCH = 200
NWB = NHALF // WBCH


def _h0_matmul(xa, W, b):
    R = 5000

    def kern(xa_ref, W_ref, b_ref, o_ref):
        o_ref[...] = jnp.maximum(
            jnp.dot(xa_ref[...], W_ref[...], preferred_element_type=jnp.float32)
            + b_ref[...][None, :], 0.0)

    return pl.pallas_call(
        kern,
        grid=(N // R,),
        in_specs=[pl.BlockSpec((R, 4), lambda i: (i, 0)),
                  pl.BlockSpec((4, H), lambda i: (0, 0)),
                  pl.BlockSpec((H,), lambda i: (0,))],
        out_specs=pl.BlockSpec((R, H), lambda i: (i, 0)),
        out_shape=jax.ShapeDtypeStruct((N, H), jnp.float32),
    )(xa, W, b)


def _edge_mlp(ea, We1, be1, We2, be2, Wl, bl):
    R = 10000
    L = Wl.shape[0]

    def kern(ea_ref, We1_ref, be1_ref, We2_ref, be2_ref, Wl_ref, bl_ref, *outs):
        t = jnp.maximum(
            jnp.dot(ea_ref[...], We1_ref[...], preferred_element_type=jnp.float32)
            + be1_ref[...][None, :], 0.0)
        e = jnp.dot(t, We2_ref[...], preferred_element_type=jnp.float32) \
            + be2_ref[...][None, :]
        for l in range(L):
            outs[l][...] = jnp.dot(e, Wl_ref[l], preferred_element_type=jnp.float32) \
                + bl_ref[l][None, :]

    return pl.pallas_call(
        kern,
        grid=(E // R,),
        in_specs=[pl.BlockSpec((R, 4), lambda i: (i, 0)),
                  pl.BlockSpec((4, H), lambda i: (0, 0)),
                  pl.BlockSpec((H,), lambda i: (0,)),
                  pl.BlockSpec((H, H), lambda i: (0, 0)),
                  pl.BlockSpec((H,), lambda i: (0,)),
                  pl.BlockSpec((L, H, H), lambda i: (0, 0, 0)),
                  pl.BlockSpec((L, H), lambda i: (0, 0))],
        out_specs=[pl.BlockSpec((R, H), lambda i: (i, 0))] * L,
        out_shape=[jax.ShapeDtypeStruct((E, H), jnp.float32)] * L,
    )(ea, We1, be1, We2, be2, Wl, bl)


def _node_update(h, aggr, Wa, ba, Wb, bb, g, bt):
    R = 5000

    def kern(h_ref, a_ref, Wa_ref, ba_ref, Wb_ref, bb_ref, g_ref, bt_ref, o_ref):
        hv = h_ref[...]
        z = hv + a_ref[...]
        z1 = jnp.maximum(
            jnp.dot(z, Wa_ref[...], preferred_element_type=jnp.float32)
            + ba_ref[...][None, :], 0.0)
        z2 = jnp.dot(z1, Wb_ref[...], preferred_element_type=jnp.float32) \
            + bb_ref[...][None, :]
        mu = jnp.mean(z2, axis=1, keepdims=True)
        d = z2 - mu
        var = jnp.mean(d * d, axis=1, keepdims=True)
        zn = d * lax.rsqrt(var + 1e-5) * g_ref[...][None, :] + bt_ref[...][None, :]
        o_ref[...] = hv + jnp.maximum(zn, 0.0)

    return pl.pallas_call(
        kern,
        grid=(N // R,),
        in_specs=[pl.BlockSpec((R, H), lambda i: (i, 0)),
                  pl.BlockSpec((R, H), lambda i: (i, 0)),
                  pl.BlockSpec((H, H), lambda i: (0, 0)),
                  pl.BlockSpec((H,), lambda i: (0,)),
                  pl.BlockSpec((H, H), lambda i: (0, 0)),
                  pl.BlockSpec((H,), lambda i: (0,)),
                  pl.BlockSpec((H,), lambda i: (0,)),
                  pl.BlockSpec((H,), lambda i: (0,))],
        out_specs=pl.BlockSpec((R, H), lambda i: (i, 0)),
        out_shape=jax.ShapeDtypeStruct((N, H), jnp.float32),
    )(h, aggr, Wa, ba, Wb, bb, g, bt)


def _pool_head(h, Wh1, bh1, Wh2, bh2):
    def kern(h_ref, W1_ref, b1_ref, W2_ref, b2_ref, o_ref):
        gs = []
        for b in range(NB):
            seg = h_ref[pl.ds(b * PER, PER), :]
            gs.append(jnp.sum(seg, axis=0) / float(PER))
        g = jnp.stack(gs, axis=0)
        q1 = jnp.maximum(
            jnp.dot(g, W1_ref[...], preferred_element_type=jnp.float32)
            + b1_ref[...][None, :], 0.0)
        q = jnp.dot(q1, W2_ref[...], preferred_element_type=jnp.float32) \
            + b2_ref[...][None, :]
        o_ref[...] = q[:, 0]

    return pl.pallas_call(
        kern,
        out_shape=jax.ShapeDtypeStruct((NB,), jnp.float32),
    )(h, Wh1, bh1, Wh2, bh2)


def _sc_aggr_body(src_ref, dst_ref, h_ref, el_ref, out_ref,
                  srcg, dstg, idxb, hb0, hb1, eb0, eb1, aggr_sh,
                  gsem0, gsem1):
    c_ax = lax.axis_index("c")
    s_ax = lax.axis_index("s")
    base_node = c_ax * NHALF

    hbs = (hb0, hb1)
    ebs = (eb0, eb1)
    sems = (gsem0, gsem1)

    def zrow(r, carry):
        for q in range(H // 16):
            hb0[r, pl.ds(q * 16, 16)] = jnp.zeros((16,), jnp.float32)
        return carry
    lax.fori_loop(0, K, zrow, 0)
    nz = jnp.where(s_ax < (NZCH % 16), NZCH // 16 + 1, NZCH // 16)

    def zchunk(i, carry):
        cid = s_ax + 16 * i
        pltpu.sync_copy(hb0.at[:, :], aggr_sh.at[pl.ds(cid * K, K), :])
        return carry
    lax.fori_loop(0, nz, zchunk, 0)
    plsc.subcore_barrier()

    tbase = s_ax * EPT

    def fire(go, ci, p):
        pltpu.async_copy(h_ref.at[srcg.at[pl.ds(ci * K, K)]], hbs[p], sems[p])
        pltpu.async_copy(
            el_ref.at[pl.ds((go + ci * K) // 2, K // 2), :], ebs[p], sems[p])

    def proc(go, ci, p):
        pltpu.make_async_copy(
            h_ref.at[srcg.at[pl.ds(ci * K, K)]], hbs[p], sems[p]).wait()
        pltpu.make_async_copy(
            el_ref.at[pl.ds((go + ci * K) // 2, K // 2), :], ebs[p], sems[p]).wait()

        def ib(v, icarry):
            d = dstg[pl.ds(ci * K + v * 16, 16)]
            dl = d - base_node
            ok = (dl >= 0) & (dl < NHALF)
            idxb[pl.ds(v * 16, 16)] = jnp.where(ok, dl, NHALF)
            return icarry
        lax.fori_loop(0, K // 16, ib, 0)
        hb, eb = hbs[p], ebs[p]

        def mrow(rp, mcarry):
            for half in range(2):
                for q in range(H // 16):
                    hb[2 * rp + half, pl.ds(q * 16, 16)] = jnp.maximum(
                        hb[2 * rp + half, pl.ds(q * 16, 16)]
                        + eb[rp, pl.ds(half * H + q * 16, 16)], 0.0)
            return mcarry
        lax.fori_loop(0, K // 2, mrow, 0)
        pltpu.sync_copy(hb, aggr_sh.at[idxb], add=True)

    def group(g, carry):
        go = tbase + g * GROUP
        pltpu.sync_copy(src_ref.at[pl.ds(go, GROUP)], srcg)
        pltpu.sync_copy(dst_ref.at[pl.ds(go, GROUP)], dstg)
        fire(go, 0, 0)

        def pair(kk, pcarry):
            cA = 2 * kk
            fire(go, cA + 1, 1)
            proc(go, cA, 0)
            fire(go, cA + 2, 0)
            proc(go, cA + 1, 1)
            return pcarry
        lax.fori_loop(0, (CPG - 3) // 2, pair, 0)
        fire(go, CPG - 2, 1)
        proc(go, CPG - 3, 0)
        fire(go, CPG - 1, 0)
        proc(go, CPG - 2, 1)
        proc(go, CPG - 1, 0)
        return carry
    lax.fori_loop(0, NGRP, group, 0)

    plsc.subcore_barrier()
    nw = jnp.where(s_ax < (NWB % 16), NWB // 16 + 1, NWB // 16)

    def wchunk(i, carry):
        cid = s_ax + 16 * i
        pltpu.sync_copy(aggr_sh.at[pl.ds(cid * WBCH, WBCH), :],
                        out_ref.at[pl.ds(base_node + cid * WBCH, WBCH), :])
        return carry
    lax.fori_loop(0, nw, wchunk, 0)


def _sc_aggregate(src, dst, h, elin):
    mesh = plsc.VectorSubcoreMesh(core_axis_name="c", subcore_axis_name="s",
                                  num_cores=2, num_subcores=16)
    f = pl.kernel(
        _sc_aggr_body,
        out_type=jax.ShapeDtypeStruct((N, H), jnp.float32),
        mesh=mesh,
        scratch_types=[
            pltpu.VMEM((GROUP,), jnp.int32),
            pltpu.VMEM((GROUP,), jnp.int32),
            pltpu.VMEM((K,), jnp.int32),
            pltpu.VMEM((K, H), jnp.float32),
            pltpu.VMEM((K, H), jnp.float32),
            pltpu.VMEM((K // 2, 2 * H), jnp.float32),
            pltpu.VMEM((K // 2, 2 * H), jnp.float32),
            pltpu.VMEM_SHARED((AGG_ROWS, H), jnp.float32),
            pltpu.SemaphoreType.DMA,
            pltpu.SemaphoreType.DMA,
        ],
        compiler_params=pltpu.CompilerParams(use_tc_tiling_on_sc=False),
    )
    return f(src, dst, h, elin)


def kernel(x, edge_index, edge_attr, batch_ids, ptr, a,
           W_np, b_np, We1, be1, We2, be2,
           conv_Wl, conv_bl, conv_Wa, conv_ba, conv_Wb, conv_bb,
           ln_g, ln_b, Wh1, bh1, Wh2, bh2):
    L = conv_Wl.shape[0]
    xa = jnp.concatenate([x, a.reshape(-1, 1)], axis=1)
    h = _h0_matmul(xa, W_np, b_np)
    elins = _edge_mlp(edge_attr, We1, be1, We2, be2, conv_Wl, conv_bl)
    src, dst = edge_index[0], edge_index[1]
    for l in range(L):
        aggr = _sc_aggregate(src, dst, h, elins[l].reshape(E // 2, 2 * H))
        h = _node_update(h, aggr, conv_Wa[l], conv_ba[l],
                         conv_Wb[l], conv_bb[l], ln_g[l], ln_b[l])
    return _pool_head(h, Wh1, bh1, Wh2, bh2)
